# baseline (speedup 1.0000x reference)
import functools

import jax
import jax.numpy as jnp
from jax import lax
from jax.experimental import pallas as pl
from jax.experimental.pallas import tpu as pltpu
from jax.experimental.pallas import tpu_sc as plsc

N_TOK = 16384
D_EMB = 2048
N_EXPERTS = 16
TOP_K = 2

N_SC = 2048
N_TC = N_TOK - N_SC

BLK = 512
N_BUF = 4
SPLITS = 2
SUB = BLK // SPLITS
NSTEPS = N_TC // BLK

NC = 2
NS = 16
NW = NC * NS
T_W = N_SC // NW
T_TILE = 16
T_P = 4
E_B = 8


def _h_copy(h_hbm, hbuf, sems, chunk, slot, s):
    return pltpu.make_async_copy(
        h_hbm.at[pl.ds(chunk * BLK + s * SUB, SUB), :],
        hbuf.at[slot, pl.ds(s * SUB, SUB), :],
        sems.at[slot, s])


def _fused_tc_kernel(h_hbm, md_ref, w1_ref, b1_ref, w2_ref, b2_ref,
                     wg_ref, bg_ref, gw_ref, idx_ref, hbuf, sems):
    i = pl.program_id(0)
    slot = jax.lax.rem(i, N_BUF)

    @pl.when(i == 0)
    def _prologue():
        for c in range(min(N_BUF - 1, NSTEPS)):
            for s in range(SPLITS):
                _h_copy(h_hbm, hbuf, sems, c, c % N_BUF, s).start()

    @pl.when(i + N_BUF - 1 < NSTEPS)
    def _prefetch():
        nxt = i + N_BUF - 1
        nslot = jax.lax.rem(nxt, N_BUF)
        for s in range(SPLITS):
            _h_copy(h_hbm, hbuf, sems, nxt, nslot, s).start()

    for s in range(SPLITS):
        _h_copy(h_hbm, hbuf, sems, i, slot, s).wait()
    hb = hbuf[slot]

    md = md_ref[...]
    g = jnp.dot(md, w1_ref[...], preferred_element_type=jnp.float32)
    g = g + b1_ref[...]
    g = 0.5 * g * (1.0 + jax.lax.erf(g * 0.7071067811865476))
    m_emb = jnp.dot(g, w2_ref[...], preferred_element_type=jnp.float32)
    m_emb = m_emb + b2_ref[...]

    logits = jnp.dot(hb, wg_ref[:D_EMB, :], preferred_element_type=jnp.float32)
    logits = logits + jnp.dot(m_emb, wg_ref[D_EMB:, :],
                              preferred_element_type=jnp.float32)
    logits = logits + bg_ref[...]

    cols = jax.lax.broadcasted_iota(jnp.int32, (BLK, N_EXPERTS), 1)
    idx1 = jnp.argmax(logits, axis=-1).astype(jnp.int32)
    v1 = jnp.max(logits, axis=-1)
    masked = jnp.where(cols == idx1[:, None], -jnp.inf, logits)
    idx2 = jnp.argmax(masked, axis=-1).astype(jnp.int32)
    v2 = jnp.max(masked, axis=-1)

    e = jnp.exp(v2 - v1)
    w2 = e / (1.0 + e)
    w1 = 1.0 - w2
    gw = jnp.where(cols == idx1[:, None], w1[:, None], 0.0)
    gw = jnp.where(cols == idx2[:, None], w2[:, None], gw)
    gw_ref[...] = gw
    idx_ref[...] = jnp.concatenate([idx1[:, None], idx2[:, None]], axis=-1)


def _bias_tc_kernel(md_ref, w1_ref, b1_ref, w2_ref, b2_ref, wgm_ref, bg_ref,
                    bias_ref):
    md = md_ref[...]
    g = jnp.dot(md, w1_ref[...], preferred_element_type=jnp.float32)
    g = g + b1_ref[...]
    g = 0.5 * g * (1.0 + jax.lax.erf(g * 0.7071067811865476))
    m_emb = jnp.dot(g, w2_ref[...], preferred_element_type=jnp.float32)
    m_emb = m_emb + b2_ref[...]
    bias_ref[...] = jnp.dot(m_emb, wgm_ref[...],
                            preferred_element_type=jnp.float32) + bg_ref[...]


def _route_tokens(chunk_v, gw_v, idx_v, lanes, t0, n_tok_loc):
    neg_inf = jnp.float32(-jnp.inf)

    def body(t, carry):
        row = chunk_v[pl.ds(t * N_EXPERTS, N_EXPERTS)]
        v1 = jnp.max(row)
        is1 = row == v1
        first1 = jnp.logical_and(is1, jnp.cumsum(is1.astype(jnp.int32)) == 1)
        masked = jnp.where(first1, neg_inf, row)
        v2 = jnp.max(masked)
        is2 = masked == v2
        first2 = jnp.logical_and(is2, jnp.cumsum(is2.astype(jnp.int32)) == 1)
        keep = jnp.logical_or(first1, first2)
        ex = jnp.where(keep, jnp.exp(row - v1), jnp.float32(0.0))
        gw_v[pl.ds(t * N_EXPERTS, N_EXPERTS)] = ex / jnp.sum(ex)
        plsc.store_compressed(idx_v.at[pl.ds(2 * t, N_EXPERTS)], lanes,
                              mask=first1)
        plsc.store_compressed(idx_v.at[pl.ds(2 * t + 1, N_EXPERTS)], lanes,
                              mask=first2)
        return carry

    lax.fori_loop(t0, t0 + n_tok_loc, body, 0)


def _sc_router_kernel(h_hbm, wgt_hbm, bias_hbm, gw_hbm, idx_hbm,
                      wg_v, h_v0, h_v1, bias_v, logit_v, gw_v, idx_v, hsem,
                      wsem, bsem, gsem):
    wid = lax.axis_index("s") * NC + lax.axis_index("c")
    tok0 = wid * T_W
    htok0 = N_TC + tok0
    pltpu.make_async_copy(wgt_hbm, wg_v, wsem).start()
    pltpu.make_async_copy(
        bias_hbm.at[pl.ds(tok0 * N_EXPERTS, T_W * N_EXPERTS)],
        bias_v, bsem).start()
    pltpu.make_async_copy(wgt_hbm, wg_v, wsem).wait()
    pltpu.make_async_copy(
        bias_hbm.at[pl.ds(tok0 * N_EXPERTS, T_W * N_EXPERTS)],
        bias_v, bsem).wait()
    lanes = lax.iota(jnp.int32, N_EXPERTS)
    zero16 = jnp.zeros((N_EXPERTS,), jnp.float32)
    n_tiles = T_W // T_TILE
    hbufs = [h_v0, h_v1]
    NCHUNK = D_EMB // 32

    def rnd2(a, b):
        ab = plsc.pack(a, b, format=plsc.PackFormat.INTERLEAVED)
        return plsc.unpack(ab, format=plsc.PackFormat.INTERLEAVED,
                           preferred_element_type=jnp.float32)

    def h_row_copy(tb, r):
        return pltpu.make_async_copy(
            h_hbm.at[htok0 + tb * T_TILE + r],
            hbufs[tb % 2].at[pl.ds(r * D_EMB, D_EMB)],
            hsem.at[tb % 2])

    for tb in range(n_tiles):
        if tb == 0:
            for r in range(T_TILE):
                h_row_copy(0, r).start()
        if tb + 1 < n_tiles:
            for r in range(T_TILE):
                h_row_copy(tb + 1, r).start()
        for r in range(T_TILE):
            h_row_copy(tb, r).wait()
        hv = hbufs[tb % 2]

        for tq in range(T_TILE // T_P):
            tbase = tq * T_P
            rows = [bias_v[pl.ds((tb * T_TILE + tbase + tt) * N_EXPERTS,
                                 N_EXPERTS)] for tt in range(T_P)]
            for eh in range(N_EXPERTS // E_B):
                def cbody(c, accs, _tbase=tbase, _eh=eh, _hv=hv):
                    hps = [rnd2(_hv[pl.ds((_tbase + tt) * D_EMB + 32 * c,
                                          16)],
                                _hv[pl.ds((_tbase + tt) * D_EMB + 32 * c
                                          + 16, 16)]) for tt in range(T_P)]
                    out = list(accs)
                    for e in range(E_B):
                        wa = wg_v[pl.ds((_eh * E_B + e) * D_EMB + 32 * c,
                                        16)]
                        wb = wg_v[pl.ds((_eh * E_B + e) * D_EMB + 32 * c
                                        + 16, 16)]
                        for tt in range(T_P):
                            ha, hb = hps[tt]
                            acc = out[tt * E_B + e] + ha * wa
                            out[tt * E_B + e] = acc + hb * wb
                    return tuple(out)

                accs = lax.fori_loop(0, NCHUNK, cbody,
                                     tuple(zero16 for _ in range(T_P * E_B)))
                for tt in range(T_P):
                    row = rows[tt]
                    for e in range(E_B):
                        sacc = jnp.sum(accs[tt * E_B + e])
                        row = row + jnp.where(lanes == (eh * E_B + e),
                                              sacc, jnp.float32(0.0))
                    rows[tt] = row
            for tt in range(T_P):
                logit_v[pl.ds((tb * T_TILE + tbase + tt) * N_EXPERTS,
                              N_EXPERTS)] = rows[tt]

    _route_tokens(logit_v, gw_v, idx_v, lanes, 0, T_W)
    pltpu.make_async_copy(
        gw_v, gw_hbm.at[pl.ds(tok0 * N_EXPERTS, T_W * N_EXPERTS)],
        gsem).start()
    pltpu.make_async_copy(
        gw_v, gw_hbm.at[pl.ds(tok0 * N_EXPERTS, T_W * N_EXPERTS)],
        gsem).wait()
    pltpu.sync_copy(idx_v.at[pl.ds(0, T_W * TOP_K)],
                    idx_hbm.at[pl.ds(tok0 * TOP_K, T_W * TOP_K)])


@functools.partial(jax.jit, static_argnames=())
def kernel(h, metadata, W1, b1, W2, b2, Wg, bg, mu):
    full = lambda shape: pl.BlockSpec(shape, lambda i: (0,) * len(shape))

    gw_tc, idx_tc = pl.pallas_call(
        _fused_tc_kernel,
        grid=(NSTEPS,),
        in_specs=[
            pl.BlockSpec(memory_space=pltpu.MemorySpace.HBM),
            pl.BlockSpec((BLK, 2), lambda i: (i, 0)),
            full((2, 16)),
            full((1, 16)),
            full((16, 8)),
            full((1, 8)),
            full((D_EMB + 8, N_EXPERTS)),
            full((1, N_EXPERTS)),
        ],
        out_specs=[
            pl.BlockSpec((BLK, N_EXPERTS), lambda i: (i, 0)),
            pl.BlockSpec((BLK, TOP_K), lambda i: (i, 0)),
        ],
        out_shape=[
            jax.ShapeDtypeStruct((N_TC, N_EXPERTS), jnp.float32),
            jax.ShapeDtypeStruct((N_TC, TOP_K), jnp.int32),
        ],
        scratch_shapes=[
            pltpu.VMEM((N_BUF, BLK, D_EMB), jnp.float32),
            pltpu.SemaphoreType.DMA((N_BUF, SPLITS)),
        ],
        compiler_params=pltpu.CompilerParams(
            dimension_semantics=("arbitrary",),
        ),
    )(h, metadata, W1, b1.reshape(1, -1), W2,
      b2.reshape(1, -1), Wg, bg.reshape(1, -1))

    bias_sc = pl.pallas_call(
        _bias_tc_kernel,
        grid=(1,),
        in_specs=[
            pl.BlockSpec((N_SC, 2), lambda i: (N_TC // N_SC, 0)),
            full((2, 16)),
            full((1, 16)),
            full((16, 8)),
            full((1, 8)),
            full((8, N_EXPERTS)),
            full((1, N_EXPERTS)),
        ],
        out_specs=pl.BlockSpec((N_SC, N_EXPERTS), lambda i: (0, 0)),
        out_shape=jax.ShapeDtypeStruct((N_SC, N_EXPERTS), jnp.float32),
    )(metadata, W1, b1.reshape(1, -1), W2, b2.reshape(1, -1),
      Wg[D_EMB:], bg.reshape(1, -1))

    sc_route = pl.kernel(
        _sc_router_kernel,
        out_type=[
            jax.ShapeDtypeStruct((N_SC * N_EXPERTS,), jnp.float32),
            jax.ShapeDtypeStruct((N_SC * TOP_K,), jnp.int32),
        ],
        mesh=plsc.VectorSubcoreMesh(core_axis_name="c", subcore_axis_name="s"),
        compiler_params=pltpu.CompilerParams(needs_layout_passes=False),
        scratch_types=[
            pltpu.VMEM((N_EXPERTS * D_EMB,), jnp.float32),
            pltpu.VMEM((T_TILE * D_EMB,), jnp.float32),
            pltpu.VMEM((T_TILE * D_EMB,), jnp.float32),
            pltpu.VMEM((T_W * N_EXPERTS,), jnp.float32),
            pltpu.VMEM((T_W * N_EXPERTS,), jnp.float32),
            pltpu.VMEM((T_W * N_EXPERTS,), jnp.float32),
            pltpu.VMEM((T_W * TOP_K + N_EXPERTS,), jnp.int32),
            pltpu.SemaphoreType.DMA((2,)),
            pltpu.SemaphoreType.DMA,
            pltpu.SemaphoreType.DMA,
            pltpu.SemaphoreType.DMA,
        ],
    )
    wgt_u = jax.lax.bitcast_convert_type(jnp.transpose(Wg[:D_EMB]),
                                         jnp.uint32)
    wgt_lsb = jnp.right_shift(wgt_u, jnp.uint32(16)) & jnp.uint32(1)
    wgt_u = (wgt_u + jnp.uint32(0x7FFF) + wgt_lsb) & jnp.uint32(0xFFFF0000)
    wgt_r = jnp.reshape(jax.lax.bitcast_convert_type(wgt_u, jnp.float32),
                        (-1,))
    gw_sc, idx_sc = sc_route(h, wgt_r, jnp.reshape(bias_sc, (-1,)))

    gw = jnp.concatenate([gw_tc, gw_sc.reshape(N_SC, N_EXPERTS)], axis=0)
    idx = jnp.concatenate([idx_tc, idx_sc.reshape(N_SC, TOP_K)], axis=0)
    return (gw, idx, mu)


# speedup vs baseline: 1.7286x; 1.7286x over previous
import functools

import jax
import jax.numpy as jnp
from jax import lax
from jax.experimental import pallas as pl
from jax.experimental.pallas import tpu as pltpu
from jax.experimental.pallas import tpu_sc as plsc

N_TOK = 16384
D_EMB = 2048
N_EXPERTS = 16
TOP_K = 2

N_SC = 1024
N_TC = N_TOK - N_SC

BLK = 512
N_BUF = 4
SPLITS = 2
SUB = BLK // SPLITS
NSTEPS = N_TC // BLK

NC = 2
NS = 16
NW = NC * NS
T_W = N_SC // NW
T_TILE = 16
T_P = 4
E_B = 8


def _h_copy(h_hbm, hbuf, sems, chunk, slot, s):
    return pltpu.make_async_copy(
        h_hbm.at[pl.ds(chunk * BLK + s * SUB, SUB), :],
        hbuf.at[slot, pl.ds(s * SUB, SUB), :],
        sems.at[slot, s])


def _fused_tc_kernel(h_hbm, md_ref, w1_ref, b1_ref, w2_ref, b2_ref,
                     wg_ref, bg_ref, gw_ref, idx_ref, hbuf, sems):
    i = pl.program_id(0)
    slot = jax.lax.rem(i, N_BUF)

    @pl.when(i == 0)
    def _prologue():
        for c in range(min(N_BUF - 1, NSTEPS)):
            for s in range(SPLITS):
                _h_copy(h_hbm, hbuf, sems, c, c % N_BUF, s).start()

    @pl.when(i + N_BUF - 1 < NSTEPS)
    def _prefetch():
        nxt = i + N_BUF - 1
        nslot = jax.lax.rem(nxt, N_BUF)
        for s in range(SPLITS):
            _h_copy(h_hbm, hbuf, sems, nxt, nslot, s).start()

    for s in range(SPLITS):
        _h_copy(h_hbm, hbuf, sems, i, slot, s).wait()
    hb = hbuf[slot]

    md = md_ref[...]
    g = jnp.dot(md, w1_ref[...], preferred_element_type=jnp.float32)
    g = g + b1_ref[...]
    g = 0.5 * g * (1.0 + jax.lax.erf(g * 0.7071067811865476))
    m_emb = jnp.dot(g, w2_ref[...], preferred_element_type=jnp.float32)
    m_emb = m_emb + b2_ref[...]

    logits = jnp.dot(hb, wg_ref[:D_EMB, :], preferred_element_type=jnp.float32)
    logits = logits + jnp.dot(m_emb, wg_ref[D_EMB:, :],
                              preferred_element_type=jnp.float32)
    logits = logits + bg_ref[...]

    cols = jax.lax.broadcasted_iota(jnp.int32, (BLK, N_EXPERTS), 1)
    idx1 = jnp.argmax(logits, axis=-1).astype(jnp.int32)
    v1 = jnp.max(logits, axis=-1)
    masked = jnp.where(cols == idx1[:, None], -jnp.inf, logits)
    idx2 = jnp.argmax(masked, axis=-1).astype(jnp.int32)
    v2 = jnp.max(masked, axis=-1)

    e = jnp.exp(v2 - v1)
    w2 = e / (1.0 + e)
    w1 = 1.0 - w2
    gw = jnp.where(cols == idx1[:, None], w1[:, None], 0.0)
    gw = jnp.where(cols == idx2[:, None], w2[:, None], gw)
    gw_ref[...] = gw
    idx_ref[...] = jnp.concatenate([idx1[:, None], idx2[:, None]], axis=-1)


def _bias_tc_kernel(md_ref, w1_ref, b1_ref, w2_ref, b2_ref, wgm_ref, bg_ref,
                    bias_ref):
    md = md_ref[...]
    g = jnp.dot(md, w1_ref[...], preferred_element_type=jnp.float32)
    g = g + b1_ref[...]
    g = 0.5 * g * (1.0 + jax.lax.erf(g * 0.7071067811865476))
    m_emb = jnp.dot(g, w2_ref[...], preferred_element_type=jnp.float32)
    m_emb = m_emb + b2_ref[...]
    bias_ref[...] = jnp.dot(m_emb, wgm_ref[...],
                            preferred_element_type=jnp.float32) + bg_ref[...]


def _route_tokens(chunk_v, gw_v, idx_v, lanes, t0, n_tok_loc):
    neg_inf = jnp.float32(-jnp.inf)

    def body(t, carry):
        row = chunk_v[pl.ds(t * N_EXPERTS, N_EXPERTS)]
        v1 = jnp.max(row)
        is1 = row == v1
        first1 = jnp.logical_and(is1, jnp.cumsum(is1.astype(jnp.int32)) == 1)
        masked = jnp.where(first1, neg_inf, row)
        v2 = jnp.max(masked)
        is2 = masked == v2
        first2 = jnp.logical_and(is2, jnp.cumsum(is2.astype(jnp.int32)) == 1)
        keep = jnp.logical_or(first1, first2)
        ex = jnp.where(keep, jnp.exp(row - v1), jnp.float32(0.0))
        gw_v[pl.ds(t * N_EXPERTS, N_EXPERTS)] = ex / jnp.sum(ex)
        plsc.store_compressed(idx_v.at[pl.ds(2 * t, N_EXPERTS)], lanes,
                              mask=first1)
        plsc.store_compressed(idx_v.at[pl.ds(2 * t + 1, N_EXPERTS)], lanes,
                              mask=first2)
        return carry

    lax.fori_loop(t0, t0 + n_tok_loc, body, 0)


def _sc_router_kernel(h_hbm, wgt_hbm, bias_hbm, gw_hbm, idx_hbm,
                      wg_v, h_v0, h_v1, bias_v, logit_v, gw_v, idx_v, hsem,
                      wsem, bsem, gsem):
    wid = lax.axis_index("s") * NC + lax.axis_index("c")
    tok0 = wid * T_W
    htok0 = N_TC + tok0
    pltpu.make_async_copy(wgt_hbm, wg_v, wsem).start()
    pltpu.make_async_copy(
        bias_hbm.at[pl.ds(tok0 * N_EXPERTS, T_W * N_EXPERTS)],
        bias_v, bsem).start()
    pltpu.make_async_copy(wgt_hbm, wg_v, wsem).wait()
    pltpu.make_async_copy(
        bias_hbm.at[pl.ds(tok0 * N_EXPERTS, T_W * N_EXPERTS)],
        bias_v, bsem).wait()
    lanes = lax.iota(jnp.int32, N_EXPERTS)
    zero16 = jnp.zeros((N_EXPERTS,), jnp.float32)
    n_tiles = T_W // T_TILE
    hbufs = [h_v0, h_v1]
    NCHUNK = D_EMB // 16

    def rnd_bf16(v):
        u = plsc.bitcast(v, jnp.uint32)
        lsb = jnp.right_shift(u, jnp.uint32(16)) & jnp.uint32(1)
        u = (u + jnp.uint32(0x7FFF) + lsb) & jnp.uint32(0xFFFF0000)
        return plsc.bitcast(u, jnp.float32)

    def h_row_copy(tb, r):
        return pltpu.make_async_copy(
            h_hbm.at[htok0 + tb * T_TILE + r],
            hbufs[tb % 2].at[pl.ds(r * D_EMB, D_EMB)],
            hsem.at[tb % 2])

    for tb in range(n_tiles):
        if tb == 0:
            for r in range(T_TILE):
                h_row_copy(0, r).start()
        if tb + 1 < n_tiles:
            for r in range(T_TILE):
                h_row_copy(tb + 1, r).start()
        for r in range(T_TILE):
            h_row_copy(tb, r).wait()
        hv = hbufs[tb % 2]

        for tq in range(T_TILE // T_P):
            tbase = tq * T_P
            rows = [bias_v[pl.ds((tb * T_TILE + tbase + tt) * N_EXPERTS,
                                 N_EXPERTS)] for tt in range(T_P)]
            for eh in range(N_EXPERTS // E_B):
                def cbody(c, accs, _tbase=tbase, _eh=eh, _hv=hv):
                    hvecs = [rnd_bf16(
                        _hv[pl.ds((_tbase + tt) * D_EMB + 16 * c, 16)])
                             for tt in range(T_P)]
                    out = list(accs)
                    for e in range(E_B):
                        wgvec = wg_v[pl.ds((_eh * E_B + e) * D_EMB + 16 * c,
                                           16)]
                        for tt in range(T_P):
                            out[tt * E_B + e] = (out[tt * E_B + e]
                                                 + hvecs[tt] * wgvec)
                    return tuple(out)

                accs = lax.fori_loop(0, NCHUNK, cbody,
                                     tuple(zero16 for _ in range(T_P * E_B)))
                for tt in range(T_P):
                    row = rows[tt]
                    for e in range(E_B):
                        sacc = jnp.sum(accs[tt * E_B + e])
                        row = row + jnp.where(lanes == (eh * E_B + e),
                                              sacc, jnp.float32(0.0))
                    rows[tt] = row
            for tt in range(T_P):
                logit_v[pl.ds((tb * T_TILE + tbase + tt) * N_EXPERTS,
                              N_EXPERTS)] = rows[tt]

    _route_tokens(logit_v, gw_v, idx_v, lanes, 0, T_W)
    pltpu.make_async_copy(
        gw_v, gw_hbm.at[pl.ds(tok0 * N_EXPERTS, T_W * N_EXPERTS)],
        gsem).start()
    pltpu.make_async_copy(
        gw_v, gw_hbm.at[pl.ds(tok0 * N_EXPERTS, T_W * N_EXPERTS)],
        gsem).wait()
    pltpu.sync_copy(idx_v.at[pl.ds(0, T_W * TOP_K)],
                    idx_hbm.at[pl.ds(tok0 * TOP_K, T_W * TOP_K)])


@functools.partial(jax.jit, static_argnames=())
def kernel(h, metadata, W1, b1, W2, b2, Wg, bg, mu):
    full = lambda shape: pl.BlockSpec(shape, lambda i: (0,) * len(shape))

    gw_tc, idx_tc = pl.pallas_call(
        _fused_tc_kernel,
        grid=(NSTEPS,),
        in_specs=[
            pl.BlockSpec(memory_space=pltpu.MemorySpace.HBM),
            pl.BlockSpec((BLK, 2), lambda i: (i, 0)),
            full((2, 16)),
            full((1, 16)),
            full((16, 8)),
            full((1, 8)),
            full((D_EMB + 8, N_EXPERTS)),
            full((1, N_EXPERTS)),
        ],
        out_specs=[
            pl.BlockSpec((BLK, N_EXPERTS), lambda i: (i, 0)),
            pl.BlockSpec((BLK, TOP_K), lambda i: (i, 0)),
        ],
        out_shape=[
            jax.ShapeDtypeStruct((N_TC, N_EXPERTS), jnp.float32),
            jax.ShapeDtypeStruct((N_TC, TOP_K), jnp.int32),
        ],
        scratch_shapes=[
            pltpu.VMEM((N_BUF, BLK, D_EMB), jnp.float32),
            pltpu.SemaphoreType.DMA((N_BUF, SPLITS)),
        ],
        compiler_params=pltpu.CompilerParams(
            dimension_semantics=("arbitrary",),
        ),
    )(h, metadata, W1, b1.reshape(1, -1), W2,
      b2.reshape(1, -1), Wg, bg.reshape(1, -1))

    bias_sc = pl.pallas_call(
        _bias_tc_kernel,
        grid=(1,),
        in_specs=[
            pl.BlockSpec((N_SC, 2), lambda i: (N_TC // N_SC, 0)),
            full((2, 16)),
            full((1, 16)),
            full((16, 8)),
            full((1, 8)),
            full((8, N_EXPERTS)),
            full((1, N_EXPERTS)),
        ],
        out_specs=pl.BlockSpec((N_SC, N_EXPERTS), lambda i: (0, 0)),
        out_shape=jax.ShapeDtypeStruct((N_SC, N_EXPERTS), jnp.float32),
    )(metadata, W1, b1.reshape(1, -1), W2, b2.reshape(1, -1),
      Wg[D_EMB:], bg.reshape(1, -1))

    sc_route = pl.kernel(
        _sc_router_kernel,
        out_type=[
            jax.ShapeDtypeStruct((N_SC * N_EXPERTS,), jnp.float32),
            jax.ShapeDtypeStruct((N_SC * TOP_K,), jnp.int32),
        ],
        mesh=plsc.VectorSubcoreMesh(core_axis_name="c", subcore_axis_name="s"),
        compiler_params=pltpu.CompilerParams(needs_layout_passes=False),
        scratch_types=[
            pltpu.VMEM((N_EXPERTS * D_EMB,), jnp.float32),
            pltpu.VMEM((T_TILE * D_EMB,), jnp.float32),
            pltpu.VMEM((T_TILE * D_EMB,), jnp.float32),
            pltpu.VMEM((T_W * N_EXPERTS,), jnp.float32),
            pltpu.VMEM((T_W * N_EXPERTS,), jnp.float32),
            pltpu.VMEM((T_W * N_EXPERTS,), jnp.float32),
            pltpu.VMEM((T_W * TOP_K + N_EXPERTS,), jnp.int32),
            pltpu.SemaphoreType.DMA((2,)),
            pltpu.SemaphoreType.DMA,
            pltpu.SemaphoreType.DMA,
            pltpu.SemaphoreType.DMA,
        ],
    )
    wgt_u = jax.lax.bitcast_convert_type(jnp.transpose(Wg[:D_EMB]),
                                         jnp.uint32)
    wgt_lsb = jnp.right_shift(wgt_u, jnp.uint32(16)) & jnp.uint32(1)
    wgt_u = (wgt_u + jnp.uint32(0x7FFF) + wgt_lsb) & jnp.uint32(0xFFFF0000)
    wgt_r = jnp.reshape(jax.lax.bitcast_convert_type(wgt_u, jnp.float32),
                        (-1,))
    gw_sc, idx_sc = sc_route(h, wgt_r, jnp.reshape(bias_sc, (-1,)))

    gw = jnp.concatenate([gw_tc, gw_sc.reshape(N_SC, N_EXPERTS)], axis=0)
    idx = jnp.concatenate([idx_tc, idx_sc.reshape(N_SC, TOP_K)], axis=0)
    return (gw, idx, mu)
